# Initial kernel scaffold; baseline (speedup 1.0000x reference)
import jax
import jax.numpy as jnp
from jax.experimental import pallas as pl


def kernel(hidden_states, gate_w, fc1_w, fc2_w):
    raise NotImplementedError("write your pallas kernel here")


# speedup vs baseline: 1.3000x; 1.3000x over previous
import jax
import jax.numpy as jnp
from jax.experimental import pallas as pl
from jax.experimental.pallas import tpu as pltpu

_T, _H, _E, _F = 128, 1024, 64, 1024


def _moe_body(x_ref, gate_ref, fc1_ref, fc2_ref, out_ref, it_s, w_s):
    e = pl.program_id(0)

    @pl.when(e == 0)
    def _router():
        logits = jnp.dot(x_ref[...], gate_ref[...],
                         preferred_element_type=jnp.float32)
        idx = jax.lax.broadcasted_iota(jnp.int32, (_T, _E), 1)
        m1 = jnp.max(logits, axis=1, keepdims=True)
        it1 = jnp.min(jnp.where(logits == m1, idx, _E), axis=1, keepdims=True)
        l2 = jnp.where(idx == it1, -jnp.inf, logits)
        m2 = jnp.max(l2, axis=1, keepdims=True)
        it2 = jnp.min(jnp.where(l2 == m2, idx, _E), axis=1, keepdims=True)
        r = jnp.exp(m2 - m1)
        w1 = 1.0 / (1.0 + r)
        it_s[:, 0:1] = it1
        it_s[:, 1:2] = it2
        w_s[:, 0:1] = w1
        w_s[:, 1:2] = 1.0 - w1

    c = jnp.sum(jnp.where(it_s[...] == e, w_s[...], 0.0),
                axis=1, keepdims=True)
    h = jnp.dot(x_ref[...], fc1_ref[0], preferred_element_type=jnp.float32)
    h = h * jax.nn.sigmoid(h) * c
    y = jnp.dot(h, fc2_ref[0], preferred_element_type=jnp.float32)

    @pl.when(e == 0)
    def _init():
        out_ref[...] = y

    @pl.when(e > 0)
    def _acc():
        out_ref[...] += y


def kernel(hidden_states, gate_w, fc1_w, fc2_w):
    return pl.pallas_call(
        _moe_body,
        grid=(_E,),
        in_specs=[
            pl.BlockSpec((_T, _H), lambda e: (0, 0)),
            pl.BlockSpec((_H, _E), lambda e: (0, 0)),
            pl.BlockSpec((1, _H, _F), lambda e: (e, 0, 0)),
            pl.BlockSpec((1, _F, _H), lambda e: (e, 0, 0)),
        ],
        out_specs=pl.BlockSpec((_T, _H), lambda e: (0, 0)),
        out_shape=jax.ShapeDtypeStruct((_T, _H), jnp.float32),
        scratch_shapes=[
            pltpu.VMEM((_T, 2), jnp.int32),
            pltpu.VMEM((_T, 2), jnp.float32),
        ],
        compiler_params=pltpu.CompilerParams(
            dimension_semantics=("arbitrary",),
        ),
    )(hidden_states, gate_w, fc1_w, fc2_w)
